# Initial kernel scaffold; baseline (speedup 1.0000x reference)
import jax
import jax.numpy as jnp
from jax.experimental import pallas as pl


def kernel(x, adj_t, adj_t2):
    raise NotImplementedError("write your pallas kernel here")


# speedup vs baseline: 3.8218x; 3.8218x over previous
import jax
import jax.numpy as jnp
from jax import lax
from jax.experimental import pallas as pl
from jax.experimental.pallas import tpu as pltpu
from jax.experimental.pallas import tpu_sc as plsc

N = 10000
NP = 10240
HALF = 64
TILES = 16
BLK = 2048
IDXW = 128
ZROWS = 128

E1, E2 = 320000, 640000
PT1 = -(-E1 // (TILES * BLK)) * BLK
PT2 = -(-E2 // (TILES * BLK)) * BLK
NB1 = PT1 // BLK
NB2 = PT2 // BLK
DUMMY = N
ACC_ROWS = 2 * NP


def _sc_body(xlo, xhi, src1, dst1, src2, dst2, zeros_hbm, out,
             zbuf, sbuf, dbuf, rows, acc, gsem):
    c = lax.axis_index("c")
    s = lax.axis_index("s")

    pltpu.sync_copy(zeros_hbm, zbuf)
    span = ACC_ROWS // TILES
    for k in range(span // ZROWS):
        pltpu.sync_copy(
            zbuf, acc.at[pl.ds(pl.multiple_of(s * span + k * ZROWS, ZROWS), ZROWS)])
    plsc.subcore_barrier()

    def process(table, src, dst, nblocks):
        rows_per_blk = BLK // IDXW

        def block(b, carry):
            r0 = pl.multiple_of((s * nblocks + b) * rows_per_blk, rows_per_blk)
            pltpu.sync_copy(src.at[pl.ds(r0, rows_per_blk)], sbuf)
            pltpu.sync_copy(dst.at[pl.ds(r0, rows_per_blk)], dbuf)
            for j in range(rows_per_blk):
                pltpu.async_copy(table.at[sbuf.at[j]], rows, gsem).wait()
                pltpu.sync_copy(rows, acc.at[dbuf.at[j]], add=True)
            return carry

        lax.fori_loop(0, nblocks, block, 0)

    @pl.when(c == 0)
    def _():
        process(xlo, src1, dst1, NB1)
        process(xlo, src2, dst2, NB2)

    @pl.when(c == 1)
    def _():
        process(xhi, src1, dst1, NB1)
        process(xhi, src2, dst2, NB2)

    plsc.subcore_barrier()

    orows = NP // TILES
    for l in range(2):
        base = pl.multiple_of(s * orows, orows)
        pltpu.sync_copy(acc.at[pl.ds(l * NP + base, orows)],
                        out.at[l, c, pl.ds(base, orows)])


@jax.jit
def kernel(x, adj_t, adj_t2):
    xlo = x[:, :HALF]
    xhi = x[:, HALF:]

    def pad_pair(src, dst, per_tile, e):
        p = TILES * per_tile - e
        src = jnp.concatenate([src, jnp.zeros((p,), jnp.int32)])
        dst = jnp.concatenate([dst, jnp.full((p,), DUMMY, jnp.int32)])
        return src.reshape(-1, IDXW), dst.reshape(-1, IDXW)

    src1, dst1 = pad_pair(adj_t[0], adj_t[1], PT1, E1)
    src2, dst2 = pad_pair(adj_t2[0], adj_t2[1] + NP, PT2, E2)
    zeros = jnp.zeros((ZROWS, HALF), jnp.float32)

    mesh = plsc.VectorSubcoreMesh(core_axis_name="c", subcore_axis_name="s",
                                  num_cores=2, num_subcores=TILES)
    run = pl.kernel(
        _sc_body,
        out_type=jax.ShapeDtypeStruct((2, 2, NP, HALF), jnp.float32),
        mesh=mesh,
        scratch_types=[
            pltpu.VMEM((ZROWS, HALF), jnp.float32),
            pltpu.VMEM((BLK // IDXW, IDXW), jnp.int32),
            pltpu.VMEM((BLK // IDXW, IDXW), jnp.int32),
            pltpu.VMEM((IDXW, HALF), jnp.float32),
            pltpu.VMEM_SHARED((ACC_ROWS, HALF), jnp.float32),
            pltpu.SemaphoreType.DMA,
        ],
        compiler_params=pltpu.CompilerParams(use_tc_tiling_on_sc=False),
    )
    out = run(xlo, xhi, src1, dst1, src2, dst2, zeros)
    return out[:, :, :N].transpose(2, 0, 1, 3).reshape(N, 4 * HALF)


# speedup vs baseline: 5.3588x; 1.4021x over previous
import jax
import jax.numpy as jnp
from jax import lax
from jax.experimental import pallas as pl
from jax.experimental.pallas import tpu as pltpu
from jax.experimental.pallas import tpu_sc as plsc

N = 10000
NP = 10240
HALF = 64
TILES = 16
BLK = 2048
IDXW = 128
ZROWS = 128
K = 2
NG = BLK // IDXW // K

E1, E2 = 320000, 640000
PT1 = -(-E1 // (TILES * BLK)) * BLK
PT2 = -(-E2 // (TILES * BLK)) * BLK
RT = (PT1 + PT2) // IDXW
NBT = (PT1 + PT2) // BLK
DUMMY = N
ACC_ROWS = 2 * NP


def _sc_body(xlo, xhi, src_all, dst_all, zeros_hbm, out,
             rows, sbuf, dbuf, acc, gsa, gsb, ssa, ssb, isem):
    c = lax.axis_index("c")
    s = lax.axis_index("s")

    pltpu.sync_copy(zeros_hbm, rows.at[0])
    span = ACC_ROWS // TILES
    for k in range(span // ZROWS):
        pltpu.sync_copy(
            rows.at[0],
            acc.at[pl.ds(pl.multiple_of(s * span + k * ZROWS, ZROWS), ZROWS)])
    plsc.subcore_barrier()

    base = pl.multiple_of(s * RT, 8)

    def tile_prog(table):
        pltpu.sync_copy(src_all.at[pl.ds(base, 16)], sbuf.at[0])
        pltpu.sync_copy(dst_all.at[pl.ds(base, 16)], dbuf.at[0])
        pltpu.async_copy(src_all.at[pl.ds(base + 16, 16)], sbuf.at[1], isem)
        pltpu.async_copy(dst_all.at[pl.ds(base + 16, 16)], dbuf.at[1], isem)
        for u in range(K):
            pltpu.async_copy(table.at[sbuf.at[0, u]], rows.at[u], gsa)

        def groups(b, par, nxt, first_block):
            for gi in range(NG):
                gp, go = (gsa, gsb) if gi % 2 == 0 else (gsb, gsa)
                sp, so = (ssa, ssb) if gi % 2 == 0 else (ssb, ssa)
                bb = (gi % 2) * K
                nb = ((gi + 1) % 2) * K
                for u in range(K):
                    pltpu.make_async_copy(
                        table.at[sbuf.at[par, 0]], rows.at[bb + u], gp).wait()
                for u in range(K):
                    pltpu.async_copy(rows.at[bb + u],
                                     acc.at[dbuf.at[par, gi * K + u]],
                                     sp, add=True)
                if not (first_block and gi == 0):
                    for u in range(K):
                        pltpu.make_async_copy(
                            rows.at[nb + u], acc.at[dbuf.at[par, 0]], so).wait()
                if gi == 0 and not first_block:
                    @pl.when(b + 1 < NBT)
                    def _():
                        r0 = base + (b + 1) * 16
                        pltpu.async_copy(src_all.at[pl.ds(r0, 16)],
                                         sbuf.at[nxt], isem)
                        pltpu.async_copy(dst_all.at[pl.ds(r0, 16)],
                                         dbuf.at[nxt], isem)
                if gi < NG - 1:
                    for u in range(K):
                        pltpu.async_copy(
                            table.at[sbuf.at[par, (gi + 1) * K + u]],
                            rows.at[nb + u], go)
                else:
                    def fire_next():
                        pltpu.make_async_copy(src_all.at[pl.ds(base, 16)],
                                              sbuf.at[nxt], isem).wait()
                        pltpu.make_async_copy(dst_all.at[pl.ds(base, 16)],
                                              dbuf.at[nxt], isem).wait()
                        for u in range(K):
                            pltpu.async_copy(table.at[sbuf.at[nxt, u]],
                                             rows.at[nb + u], go)
                    if first_block:
                        fire_next()
                    else:
                        pl.when(b + 1 < NBT)(fire_next)

        groups(0, 0, 1, True)

        def block(b, carry):
            par = lax.rem(b, 2)
            groups(b, par, 1 - par, False)
            return carry

        lax.fori_loop(1, NBT, block, 0)

        for u in range(K):
            pltpu.make_async_copy(rows.at[K + u], acc.at[dbuf.at[0, 0]],
                                  ssb).wait()

    @pl.when(c == 0)
    def _():
        tile_prog(xlo)

    @pl.when(c == 1)
    def _():
        tile_prog(xhi)

    plsc.subcore_barrier()

    orows = NP // TILES
    for l in range(2):
        ob = pl.multiple_of(s * orows, orows)
        pltpu.sync_copy(acc.at[pl.ds(l * NP + ob, orows)],
                        out.at[l, c, pl.ds(ob, orows)])


@jax.jit
def kernel(x, adj_t, adj_t2):
    xlo = x[:, :HALF]
    xhi = x[:, HALF:]

    def pad_list(src, dst, per_tile, e, dst_off):
        p = TILES * per_tile - e
        src = jnp.concatenate([src, jnp.zeros((p,), jnp.int32)])
        dst = jnp.concatenate([dst + dst_off, jnp.full((p,), DUMMY, jnp.int32)])
        return src.reshape(TILES, per_tile), dst.reshape(TILES, per_tile)

    s1, d1 = pad_list(adj_t[0], adj_t[1], PT1, E1, 0)
    s2, d2 = pad_list(adj_t2[0], adj_t2[1], PT2, E2, NP)
    src_all = jnp.concatenate([s1, s2], axis=1).reshape(-1, IDXW)
    dst_all = jnp.concatenate([d1, d2], axis=1).reshape(-1, IDXW)
    zeros = jnp.zeros((ZROWS, HALF), jnp.float32)

    mesh = plsc.VectorSubcoreMesh(core_axis_name="c", subcore_axis_name="s",
                                  num_cores=2, num_subcores=TILES)
    run = pl.kernel(
        _sc_body,
        out_type=jax.ShapeDtypeStruct((2, 2, NP, HALF), jnp.float32),
        mesh=mesh,
        scratch_types=[
            pltpu.VMEM((2 * K, IDXW, HALF), jnp.float32),
            pltpu.VMEM((2, BLK // IDXW, IDXW), jnp.int32),
            pltpu.VMEM((2, BLK // IDXW, IDXW), jnp.int32),
            pltpu.VMEM_SHARED((ACC_ROWS, HALF), jnp.float32),
            pltpu.SemaphoreType.DMA,
            pltpu.SemaphoreType.DMA,
            pltpu.SemaphoreType.DMA,
            pltpu.SemaphoreType.DMA,
            pltpu.SemaphoreType.DMA,
        ],
        compiler_params=pltpu.CompilerParams(use_tc_tiling_on_sc=False),
    )
    out = run(xlo, xhi, src_all, dst_all, zeros)
    return out[:, :, :N].transpose(2, 0, 1, 3).reshape(N, 4 * HALF)
